# baseline (speedup 1.0000x reference)
import functools

import jax
import jax.numpy as jnp
from jax import lax
from jax.experimental import pallas as pl
from jax.experimental.pallas import tpu as pltpu
from jax.experimental.pallas import tpu_sc as plsc

B = 16384
D = 32
NC = 2
NS = 16
NW = NC * NS
BPW = B // NW
CH = 128
NCHUNK = BPW // CH


def _sc_gather_body(uids, iids, gu_t, gi_t, mu_t, mi_t,
                    gu_o, gi_o, mu_o, mi_o,
                    uidx, iidx, rows_a, rows_b, sem_a, sem_b):
    wid = lax.axis_index("s") * NC + lax.axis_index("c")
    base = wid * BPW
    row0 = wid * NCHUNK

    pltpu.sync_copy(uids.at[pl.ds(row0, NCHUNK)], uidx)
    pltpu.sync_copy(iids.at[pl.ds(row0, NCHUNK)], iidx)

    def fire(table, idx, rows, sem):
        cps = []
        for j in range(NCHUNK):
            cps.append(pltpu.async_copy(
                table.at[idx.at[j]], rows.at[pl.ds(j * CH, CH)], sem))
        return cps

    def drain(cps):
        for cp in cps:
            cp.wait()

    cps_a = fire(gu_t, uidx, rows_a, sem_a)
    cps_b = fire(mu_t, uidx, rows_b, sem_b)
    drain(cps_a)
    pltpu.sync_copy(rows_a, gu_o.at[pl.ds(base, BPW)])
    cps_a = fire(gi_t, iidx, rows_a, sem_a)
    drain(cps_b)
    pltpu.sync_copy(rows_b, mu_o.at[pl.ds(base, BPW)])
    cps_b = fire(mi_t, iidx, rows_b, sem_b)
    drain(cps_a)
    pltpu.sync_copy(rows_a, gi_o.at[pl.ds(base, BPW)])
    drain(cps_b)
    pltpu.sync_copy(rows_b, mi_o.at[pl.ds(base, BPW)])


@functools.cache
def _sc_gather():
    return pl.kernel(
        _sc_gather_body,
        mesh=plsc.VectorSubcoreMesh(core_axis_name="c", subcore_axis_name="s",
                                    num_cores=NC, num_subcores=NS),
        out_type=[jax.ShapeDtypeStruct((B, D), jnp.float32)] * 4,
        scratch_types=[
            pltpu.VMEM((NCHUNK, CH), jnp.int32),
            pltpu.VMEM((NCHUNK, CH), jnp.int32),
            pltpu.VMEM((BPW, D), jnp.float32),
            pltpu.VMEM((BPW, D), jnp.float32),
            pltpu.SemaphoreType.DMA,
            pltpu.SemaphoreType.DMA,
        ],
        compiler_params=pltpu.CompilerParams(use_tc_tiling_on_sc=False),
    )


BLK = 2048


def _tc_dense_body(gu_r, gi_r, mu_r, mi_r, w1u_r, w1i_r, b1_r, w2_r, b2_r,
                   w3_r, b3_r, wfg_r, wfh_r, bf_r, o_r):
    f32 = jnp.float32
    h = mu_r[...] @ w1u_r[...] + mi_r[...] @ w1i_r[...] + b1_r[...]
    h = jnp.maximum(h, 0.0)
    h = jnp.maximum(h @ w2_r[...] + b2_r[...], 0.0)
    h = jnp.maximum(h @ w3_r[...] + b3_r[...], 0.0)
    g = gu_r[...] * gi_r[...]
    z = (jnp.sum(g * wfg_r[...], axis=1, dtype=f32)
         + jnp.sum(h * wfh_r[...], axis=1, dtype=f32) + bf_r[0, 0])
    o_r[...] = jax.nn.sigmoid(z).reshape(1, 1, BLK)


def _tc_dense(gu, gi, mu, mi, w1u, w1i, b1, w2, b2, w3, b3, wfg, wfh, bf):
    nblk = B // BLK
    row_spec = pl.BlockSpec((BLK, D), lambda i: (i, 0))
    full = lambda s: pl.BlockSpec(s, lambda i: (0,) * len(s))
    out = pl.pallas_call(
        _tc_dense_body,
        grid=(nblk,),
        in_specs=[
            row_spec, row_spec, row_spec, row_spec,
            full((D, 64)), full((D, 64)), full((1, 64)),
            full((64, 32)), full((1, 32)),
            full((32, 16)), full((1, 16)),
            full((1, D)), full((1, 16)), full((1, 1)),
        ],
        out_specs=pl.BlockSpec((1, 1, BLK), lambda i: (i, 0, 0)),
        out_shape=jax.ShapeDtypeStruct((nblk, 1, BLK), jnp.float32),
    )(gu, gi, mu, mi, w1u, w1i, b1, w2, b2, w3, b3, wfg, wfh, bf)
    return out.reshape(-1)


def kernel(user_ids, item_ids, gmf_user_table, gmf_item_table,
           mlp_user_table, mlp_item_table, W1, b1, W2, b2, W3, b3, Wf, bf):
    uid = user_ids.astype(jnp.int32).reshape(B // CH, CH)
    iid = item_ids.astype(jnp.int32).reshape(B // CH, CH)
    gu, gi, mu, mi = _sc_gather()(
        uid, iid, gmf_user_table, gmf_item_table,
        mlp_user_table, mlp_item_table)
    return _tc_dense(
        gu, gi, mu, mi,
        W1[:D], W1[D:], b1.reshape(1, -1),
        W2, b2.reshape(1, -1),
        W3, b3.reshape(1, -1),
        Wf[:D, 0].reshape(1, D), Wf[D:, 0].reshape(1, 16),
        bf.reshape(1, 1))


# speedup vs baseline: 1.4152x; 1.4152x over previous
import functools

import jax
import jax.numpy as jnp
from jax import lax
from jax.experimental import pallas as pl
from jax.experimental.pallas import tpu as pltpu
from jax.experimental.pallas import tpu_sc as plsc

B = 16384
D = 32
NC = 2
NS = 16
NW = NC * NS
BPW = B // NW
RJ = BPW // 2


def _sc_gather_body(uids, iids, gu_t, gi_t, mu_t, mi_t,
                    gu_o, gi_o, mu_o, mi_o,
                    uidx, iidx, rows_a, rows_b, sem_a, sem_b):
    wid = lax.axis_index("s") * NC + lax.axis_index("c")
    base = wid * BPW

    pltpu.sync_copy(uids.at[pl.ds(base, BPW)], uidx)
    pltpu.sync_copy(iids.at[pl.ds(base, BPW)], iidx)

    def fire(table, idx_v, off, rows, sem):
        def body(j, carry):
            v = idx_v[pl.ds(off + j * 16, 16)]
            for l in range(16):
                pltpu.async_copy(table.at[pl.ds(v[l], 1)],
                                 rows.at[pl.ds(j * 16 + l, 1)], sem)
            return carry
        lax.fori_loop(0, RJ // 16, body, 0, unroll=False)

    def drain(table, rows, sem):
        pltpu.make_async_copy(table.at[pl.ds(0, RJ)], rows, sem).wait()

    jobs = [(gu_t, uidx, 0, gu_o), (gu_t, uidx, RJ, gu_o),
            (mu_t, uidx, 0, mu_o), (mu_t, uidx, RJ, mu_o),
            (gi_t, iidx, 0, gi_o), (gi_t, iidx, RJ, gi_o),
            (mi_t, iidx, 0, mi_o), (mi_t, iidx, RJ, mi_o)]
    bufs = [(rows_a, sem_a), (rows_b, sem_b)]
    fire(jobs[0][0], jobs[0][1], jobs[0][2], *bufs[0])
    fire(jobs[1][0], jobs[1][1], jobs[1][2], *bufs[1])
    for k, (table, idx_v, off, out) in enumerate(jobs):
        rows, sem = bufs[k % 2]
        drain(table, rows, sem)
        pltpu.sync_copy(rows, out.at[pl.ds(base + off, RJ)])
        if k + 2 < len(jobs):
            nt, nidx, noff, _ = jobs[k + 2]
            fire(nt, nidx, noff, rows, sem)


@functools.cache
def _sc_gather():
    return pl.kernel(
        _sc_gather_body,
        mesh=plsc.VectorSubcoreMesh(core_axis_name="c", subcore_axis_name="s",
                                    num_cores=NC, num_subcores=NS),
        out_type=[jax.ShapeDtypeStruct((B, D), jnp.float32)] * 4,
        scratch_types=[
            pltpu.VMEM((BPW,), jnp.int32),
            pltpu.VMEM((BPW,), jnp.int32),
            pltpu.VMEM((RJ, D), jnp.float32),
            pltpu.VMEM((RJ, D), jnp.float32),
            pltpu.SemaphoreType.DMA,
            pltpu.SemaphoreType.DMA,
        ],
    )


BLK = 2048


def _tc_dense_body(gu_r, gi_r, mu_r, mi_r, w1u_r, w1i_r, b1_r, w2_r, b2_r,
                   w3_r, b3_r, wfg_r, wfh_r, bf_r, o_r):
    f32 = jnp.float32
    h = mu_r[...] @ w1u_r[...] + mi_r[...] @ w1i_r[...] + b1_r[...]
    h = jnp.maximum(h, 0.0)
    h = jnp.maximum(h @ w2_r[...] + b2_r[...], 0.0)
    h = jnp.maximum(h @ w3_r[...] + b3_r[...], 0.0)
    g = gu_r[...] * gi_r[...]
    z = (jnp.sum(g * wfg_r[...], axis=1, dtype=f32)
         + jnp.sum(h * wfh_r[...], axis=1, dtype=f32) + bf_r[0, 0])
    o_r[...] = jax.nn.sigmoid(z).reshape(1, 1, BLK)


def _tc_dense(gu, gi, mu, mi, w1u, w1i, b1, w2, b2, w3, b3, wfg, wfh, bf):
    nblk = B // BLK
    row_spec = pl.BlockSpec((BLK, D), lambda i: (i, 0))
    full = lambda s: pl.BlockSpec(s, lambda i: (0,) * len(s))
    out = pl.pallas_call(
        _tc_dense_body,
        grid=(nblk,),
        in_specs=[
            row_spec, row_spec, row_spec, row_spec,
            full((D, 64)), full((D, 64)), full((1, 64)),
            full((64, 32)), full((1, 32)),
            full((32, 16)), full((1, 16)),
            full((1, D)), full((1, 16)), full((1, 1)),
        ],
        out_specs=pl.BlockSpec((1, 1, BLK), lambda i: (i, 0, 0)),
        out_shape=jax.ShapeDtypeStruct((nblk, 1, BLK), jnp.float32),
    )(gu, gi, mu, mi, w1u, w1i, b1, w2, b2, w3, b3, wfg, wfh, bf)
    return out.reshape(-1)


def kernel(user_ids, item_ids, gmf_user_table, gmf_item_table,
           mlp_user_table, mlp_item_table, W1, b1, W2, b2, W3, b3, Wf, bf):
    uid = user_ids.astype(jnp.int32)
    iid = item_ids.astype(jnp.int32)
    gu, gi, mu, mi = _sc_gather()(
        uid, iid, gmf_user_table, gmf_item_table,
        mlp_user_table, mlp_item_table)
    return _tc_dense(
        gu, gi, mu, mi,
        W1[:D], W1[D:], b1.reshape(1, -1),
        W2, b2.reshape(1, -1),
        W3, b3.reshape(1, -1),
        Wf[:D, 0].reshape(1, D), Wf[D:, 0].reshape(1, 16),
        bf.reshape(1, 1))


# speedup vs baseline: 3.1840x; 2.2498x over previous
import functools

import jax
import jax.numpy as jnp
from jax import lax
from jax.experimental import pallas as pl
from jax.experimental.pallas import tpu as pltpu
from jax.experimental.pallas import tpu_sc as plsc

B = 16384
D = 32
V = 1000001
NC = 2
NS = 16
NW = NC * NS
BPW = B // NW
NBANK = 8


def _sc_gather_body(uids, iids, gu_t, gi_t, mu_t, mi_t,
                    gu_o, gi_o, mu_o, mi_o,
                    uidx, iidx, blk, rows, sem_a, sem_b):
    wid = lax.axis_index("s") * NC + lax.axis_index("c")
    base = wid * BPW

    pltpu.sync_copy(uids.at[pl.ds(base, BPW)], uidx)
    pltpu.sync_copy(iids.at[pl.ds(base, BPW)], iidx)

    iota16 = jnp.arange(16, dtype=jnp.int32)
    lo = iota16
    hi = iota16 + 16

    def fire(table, idx_scalar, slot, sem):
        col = pl.multiple_of((idx_scalar >> 7) << 7, 128)
        pltpu.async_copy(table.at[:, pl.ds(col, 128)], blk.at[slot], sem)

    def drain_one(table, slot, sem):
        pltpu.make_async_copy(
            table.at[:, pl.ds(0, 128)], blk.at[slot], sem).wait()

    def extract(r, idx_scalar, slot):
        lane = jnp.full((16,), idx_scalar & 127, dtype=jnp.int32)
        rr = jnp.full((16,), r, dtype=jnp.int32)
        x0 = plsc.load_gather(blk.at[slot], [lo, lane])
        x1 = plsc.load_gather(blk.at[slot], [hi, lane])
        plsc.store_scatter(rows, [lo, rr], x0)
        plsc.store_scatter(rows, [hi, rr], x1)

    def gather_table(table, idx_v, out):
        def body(g, carry):
            v = idx_v[pl.ds(g * 16, 16)]
            for b in range(NBANK):
                fire(table, v[b], b, sem_a)
            for b in range(NBANK):
                fire(table, v[NBANK + b], NBANK + b, sem_b)
            for b in range(NBANK):
                drain_one(table, b, sem_a)
            for b in range(NBANK):
                extract(g * 16 + b, v[b], b)
            for b in range(NBANK):
                drain_one(table, NBANK + b, sem_b)
            for b in range(NBANK):
                extract(g * 16 + NBANK + b, v[NBANK + b], NBANK + b)
            return carry
        lax.fori_loop(0, BPW // 16, body, 0, unroll=False)
        pltpu.sync_copy(rows, out.at[:, pl.ds(base, BPW)])

    gather_table(gu_t, uidx, gu_o)
    gather_table(mu_t, uidx, mu_o)
    gather_table(gi_t, iidx, gi_o)
    gather_table(mi_t, iidx, mi_o)


@functools.cache
def _sc_gather():
    return pl.kernel(
        _sc_gather_body,
        mesh=plsc.VectorSubcoreMesh(core_axis_name="c", subcore_axis_name="s",
                                    num_cores=NC, num_subcores=NS),
        out_type=[jax.ShapeDtypeStruct((D, B), jnp.float32)] * 4,
        scratch_types=[
            pltpu.VMEM((BPW,), jnp.int32),
            pltpu.VMEM((BPW,), jnp.int32),
            pltpu.VMEM((2 * NBANK, D, 128), jnp.float32),
            pltpu.VMEM((D, BPW), jnp.float32),
            pltpu.SemaphoreType.DMA,
            pltpu.SemaphoreType.DMA,
        ],
        compiler_params=pltpu.CompilerParams(needs_layout_passes=False),
    )


BLK = 2048


def _tc_dense_body(gu_r, gi_r, mu_r, mi_r, w1u_r, w1i_r, b1_r, w2_r, b2_r,
                   w3_r, b3_r, wfg_r, wfh_r, bf_r, o_r):
    f32 = jnp.float32
    h = w1u_r[...] @ mu_r[...] + w1i_r[...] @ mi_r[...] + b1_r[...]
    h = jnp.maximum(h, 0.0)
    h = jnp.maximum(w2_r[...] @ h + b2_r[...], 0.0)
    h = jnp.maximum(w3_r[...] @ h + b3_r[...], 0.0)
    g = gu_r[...] * gi_r[...]
    z = (jnp.sum(g * wfg_r[...], axis=0, dtype=f32)
         + jnp.sum(h * wfh_r[...], axis=0, dtype=f32) + bf_r[0, 0])
    o_r[...] = jax.nn.sigmoid(z).reshape(1, 1, BLK)


def _tc_dense(gu, gi, mu, mi, w1u, w1i, b1, w2, b2, w3, b3, wfg, wfh, bf):
    nblk = B // BLK
    col_spec = pl.BlockSpec((D, BLK), lambda i: (0, i))
    full = lambda s: pl.BlockSpec(s, lambda i: (0,) * len(s))
    out = pl.pallas_call(
        _tc_dense_body,
        grid=(nblk,),
        in_specs=[
            col_spec, col_spec, col_spec, col_spec,
            full((64, D)), full((64, D)), full((64, 1)),
            full((32, 64)), full((32, 1)),
            full((16, 32)), full((16, 1)),
            full((D, 1)), full((16, 1)), full((1, 1)),
        ],
        out_specs=pl.BlockSpec((1, 1, BLK), lambda i: (i, 0, 0)),
        out_shape=jax.ShapeDtypeStruct((nblk, 1, BLK), jnp.float32),
    )(gu, gi, mu, mi, w1u, w1i, b1, w2, b2, w3, b3, wfg, wfh, bf)
    return out.reshape(-1)


def kernel(user_ids, item_ids, gmf_user_table, gmf_item_table,
           mlp_user_table, mlp_item_table, W1, b1, W2, b2, W3, b3, Wf, bf):
    uid = user_ids.astype(jnp.int32)
    iid = item_ids.astype(jnp.int32)
    gu, gi, mu, mi = _sc_gather()(
        uid, iid, gmf_user_table.T, gmf_item_table.T,
        mlp_user_table.T, mlp_item_table.T)
    return _tc_dense(
        gu, gi, mu, mi,
        W1[:D].T, W1[D:].T, b1.reshape(-1, 1),
        W2.T, b2.reshape(-1, 1),
        W3.T, b3.reshape(-1, 1),
        Wf[:D], Wf[D:],
        bf.reshape(1, 1))


# speedup vs baseline: 3.2106x; 1.0084x over previous
import functools

import jax
import jax.numpy as jnp
from jax import lax
from jax.experimental import pallas as pl
from jax.experimental.pallas import tpu as pltpu
from jax.experimental.pallas import tpu_sc as plsc

B = 16384
D = 32
V = 1000001
NC = 2
NS = 16
NW = NC * NS
BPW = B // NW
NBANK = 8


def _sc_gather_body(uids, iids, gu_t, gi_t, mu_t, mi_t,
                    gu_o, gi_o, mu_o, mi_o,
                    uidx, iidx, blk, rows, sem_a, sem_b):
    wid = lax.axis_index("s") * NC + lax.axis_index("c")
    base = wid * BPW

    pltpu.sync_copy(uids.at[pl.ds(base, BPW)], uidx)
    pltpu.sync_copy(iids.at[pl.ds(base, BPW)], iidx)

    iota16 = jnp.arange(16, dtype=jnp.int32)
    lo = iota16
    hi = iota16 + 16

    sub = (iota16 >> 3) & 3
    subhi = ((iota16 + 16) >> 3) & 3
    feat = iota16 & 7

    def fire(table, idx_scalar, slot, sem):
        col = pl.multiple_of((idx_scalar >> 7) << 7, 128)
        for a in range(4):
            pltpu.async_copy(table.at[a, :, pl.ds(col, 128)],
                             blk.at[slot, a], sem)

    def drain_one(table, slot, sem):
        pltpu.make_async_copy(
            table.at[:, :, pl.ds(0, 128)], blk.at[slot], sem).wait()

    def extract(r, idx_scalar, slot):
        lane = jnp.full((16,), idx_scalar & 127, dtype=jnp.int32)
        rr = jnp.full((16,), r, dtype=jnp.int32)
        x0 = plsc.load_gather(blk.at[slot], [sub, feat, lane])
        x1 = plsc.load_gather(blk.at[slot], [subhi, feat, lane])
        plsc.store_scatter(rows, [lo, rr], x0)
        plsc.store_scatter(rows, [hi, rr], x1)

    def gather_table(table, idx_v, out):
        def body(g, carry):
            v = idx_v[pl.ds(g * 16, 16)]
            for b in range(NBANK):
                fire(table, v[b], b, sem_a)
            for b in range(NBANK):
                fire(table, v[NBANK + b], NBANK + b, sem_b)
            for b in range(NBANK):
                drain_one(table, b, sem_a)
            for b in range(NBANK):
                extract(g * 16 + b, v[b], b)
            for b in range(NBANK):
                drain_one(table, NBANK + b, sem_b)
            for b in range(NBANK):
                extract(g * 16 + NBANK + b, v[NBANK + b], NBANK + b)
            return carry
        lax.fori_loop(0, BPW // 16, body, 0, unroll=False)
        pltpu.sync_copy(rows, out.at[:, pl.ds(base, BPW)])

    gather_table(gu_t, uidx, gu_o)
    gather_table(mu_t, uidx, mu_o)
    gather_table(gi_t, iidx, gi_o)
    gather_table(mi_t, iidx, mi_o)


@functools.cache
def _sc_gather():
    return pl.kernel(
        _sc_gather_body,
        mesh=plsc.VectorSubcoreMesh(core_axis_name="c", subcore_axis_name="s",
                                    num_cores=NC, num_subcores=NS),
        out_type=[jax.ShapeDtypeStruct((D, B), jnp.float32)] * 4,
        scratch_types=[
            pltpu.VMEM((BPW,), jnp.int32),
            pltpu.VMEM((BPW,), jnp.int32),
            pltpu.VMEM((2 * NBANK, 4, 8, 128), jnp.float32),
            pltpu.VMEM((D, BPW), jnp.float32),
            pltpu.SemaphoreType.DMA,
            pltpu.SemaphoreType.DMA,
        ],
        compiler_params=pltpu.CompilerParams(needs_layout_passes=False),
    )


BLK = 2048


def _tc_dense_body(gu_r, gi_r, mu_r, mi_r, w1u_r, w1i_r, b1_r, w2_r, b2_r,
                   w3_r, b3_r, wfg_r, wfh_r, bf_r, o_r):
    f32 = jnp.float32
    h = w1u_r[...] @ mu_r[...] + w1i_r[...] @ mi_r[...] + b1_r[...]
    h = jnp.maximum(h, 0.0)
    h = jnp.maximum(w2_r[...] @ h + b2_r[...], 0.0)
    h = jnp.maximum(w3_r[...] @ h + b3_r[...], 0.0)
    g = gu_r[...] * gi_r[...]
    z = (jnp.sum(g * wfg_r[...], axis=0, dtype=f32)
         + jnp.sum(h * wfh_r[...], axis=0, dtype=f32) + bf_r[0, 0])
    o_r[...] = jax.nn.sigmoid(z).reshape(1, 1, BLK)


def _tc_dense(gu, gi, mu, mi, w1u, w1i, b1, w2, b2, w3, b3, wfg, wfh, bf):
    nblk = B // BLK
    col_spec = pl.BlockSpec((D, BLK), lambda i: (0, i))
    full = lambda s: pl.BlockSpec(s, lambda i: (0,) * len(s))
    out = pl.pallas_call(
        _tc_dense_body,
        grid=(nblk,),
        in_specs=[
            col_spec, col_spec, col_spec, col_spec,
            full((64, D)), full((64, D)), full((64, 1)),
            full((32, 64)), full((32, 1)),
            full((16, 32)), full((16, 1)),
            full((D, 1)), full((16, 1)), full((1, 1)),
        ],
        out_specs=pl.BlockSpec((1, 1, BLK), lambda i: (i, 0, 0)),
        out_shape=jax.ShapeDtypeStruct((nblk, 1, BLK), jnp.float32),
    )(gu, gi, mu, mi, w1u, w1i, b1, w2, b2, w3, b3, wfg, wfh, bf)
    return out.reshape(-1)


def kernel(user_ids, item_ids, gmf_user_table, gmf_item_table,
           mlp_user_table, mlp_item_table, W1, b1, W2, b2, W3, b3, Wf, bf):
    uid = user_ids.astype(jnp.int32)
    iid = item_ids.astype(jnp.int32)
    t3 = lambda t: t.T.reshape(4, 8, V)
    gu, gi, mu, mi = _sc_gather()(
        uid, iid, t3(gmf_user_table), t3(gmf_item_table),
        t3(mlp_user_table), t3(mlp_item_table))
    return _tc_dense(
        gu, gi, mu, mi,
        W1[:D].T, W1[D:].T, b1.reshape(-1, 1),
        W2.T, b2.reshape(-1, 1),
        W3.T, b3.reshape(-1, 1),
        Wf[:D], Wf[D:],
        bf.reshape(1, 1))
